# baseline (speedup 1.0000x reference)
import jax
import jax.numpy as jnp
from jax import lax
from jax.experimental import pallas as pl
from jax.experimental.pallas import tpu as pltpu
from jax.experimental.pallas import tpu_sc as plsc

R1, N1, N2, R2 = 4, 1000, 1000, 4
B = 1024
NC, NS, L = 2, 16, 16
NW = NC * NS
RPW = B // NW
K = 8
G = (B * R2) // L
W = N2 * R2


def _body(row_h, col_h, t3_h, out_h, rbuf, cbuf, ridx, cidx, ibuf, obuf):
    wid = lax.axis_index("s") * NC + lax.axis_index("c")
    i0 = wid * RPW

    pltpu.sync_copy(row_h.at[pl.ds(i0, RPW)], rbuf)
    pltpu.sync_copy(col_h, cbuf)

    iota = lax.iota(jnp.int32, L)
    div4 = lax.shift_right_logical(iota, 2)
    r2pat = jnp.bitwise_and(iota, 3)

    for r1 in range(R1):
        for h in range(RPW // L):
            ridx[pl.ds(r1 * RPW + h * L, L)] = rbuf[pl.ds(h * L, L)] + r1 * N1

    def cbody(g, carry):
        cidx[pl.ds(g * L, L)] = plsc.load_gather(cbuf, [div4 + g * 4]) * R2 + r2pat
        return carry

    lax.fori_loop(0, G, cbody, 0)

    for r1 in range(R1):
        for cc in range(RPW // K):
            off = r1 * RPW + cc * K
            pltpu.sync_copy(t3_h.at[ridx.at[pl.ds(off, K)]], ibuf)

            def gbody(g, carry):
                cvec = cidx[pl.ds(g * L, L)]
                for s in range(K):
                    obuf[s, pl.ds(g * L, L)] = plsc.load_gather(ibuf.at[s], [cvec])
                return carry

            lax.fori_loop(0, G, gbody, 0)
            orow = r1 * B + i0 + cc * K
            pltpu.sync_copy(obuf, out_h.at[pl.ds(orow, K)])


@jax.jit
def _tt_gather(row, col, t3):
    mesh = plsc.VectorSubcoreMesh(core_axis_name="c", subcore_axis_name="s")
    f = pl.kernel(
        _body,
        mesh=mesh,
        out_type=jax.ShapeDtypeStruct((R1 * B, B * R2), jnp.float32),
        compiler_params=pltpu.CompilerParams(
            needs_layout_passes=False, use_tc_tiling_on_sc=False
        ),
        scratch_types=[
            pltpu.VMEM((RPW,), jnp.int32),
            pltpu.VMEM((B,), jnp.int32),
            pltpu.VMEM((R1 * RPW,), jnp.int32),
            pltpu.VMEM((B * R2,), jnp.int32),
            pltpu.VMEM((K, W), jnp.float32),
            pltpu.VMEM((K, B * R2), jnp.float32),
        ],
    )
    return f(row, col, t3)


def kernel(indices, TT_core):
    row = indices[:, 0].astype(jnp.int32)
    col = indices[:, 1].astype(jnp.int32)
    t3 = TT_core.reshape(R1 * N1, W)
    out = _tt_gather(row, col, t3)
    return out.reshape(R1, B, B, R2)


# speedup vs baseline: 1.1439x; 1.1439x over previous
import jax
import jax.numpy as jnp
from jax import lax
from jax.experimental import pallas as pl
from jax.experimental.pallas import tpu as pltpu
from jax.experimental.pallas import tpu_sc as plsc

R1, N1, N2, R2 = 4, 1000, 1000, 4
B = 1024
NC, NS, L = 2, 16, 16
NW = NC * NS
RPW = B // NW
K = 8
KH = K // 2
G = (B * R2) // L
W = N2 * R2
NCHUNK = R1 * RPW // K


def _body(row_h, col_h, t3_h, out_h, rbuf, cbuf, ridx, cidx, ibuf, obuf,
          isem0, isem1, osem0, osem1):
    wid = lax.axis_index("s") * NC + lax.axis_index("c")
    i0 = wid * RPW
    isems = (isem0, isem1)
    osems = (osem0, osem1)

    pltpu.sync_copy(row_h.at[pl.ds(i0, RPW)], rbuf)

    iota = lax.iota(jnp.int32, L)
    div4 = lax.shift_right_logical(iota, 2)
    r2pat = jnp.bitwise_and(iota, 3)

    for r1 in range(R1):
        for h in range(RPW // L):
            ridx[pl.ds(r1 * RPW + h * L, L)] = rbuf[pl.ds(h * L, L)] + r1 * N1

    def start_in(ci, slot):
        return pltpu.async_copy(
            t3_h.at[ridx.at[pl.ds(ci * K, K)]], ibuf.at[slot], isems[slot]
        )

    in_h = [None, None]
    in_h[0] = start_in(0, 0)

    pltpu.sync_copy(col_h, cbuf)

    @plsc.parallel_loop(0, G, unroll=4)
    def _cbody(g):
        cidx[pl.ds(g * L, L)] = plsc.load_gather(cbuf, [div4 + g * 4]) * R2 + r2pat

    out_h_pend = [None, None]
    for ci in range(NCHUNK):
        slot = ci % 2
        if ci + 1 < NCHUNK:
            in_h[1 - slot] = start_in(ci + 1, 1 - slot)
        in_h[slot].wait()
        r1, cc = ci // (RPW // K), ci % (RPW // K)
        for hh in range(2):
            if out_h_pend[hh] is not None:
                out_h_pend[hh].wait()
            src = ibuf.at[slot]
            dst = obuf.at[hh]

            @plsc.parallel_loop(0, G, unroll=2)
            def _gbody(g):
                cvec = cidx[pl.ds(g * L, L)]
                for s in range(KH):
                    dst[s, pl.ds(g * L, L)] = plsc.load_gather(
                        src.at[hh * KH + s], [cvec]
                    )

            orow = r1 * B + i0 + cc * K + hh * KH
            out_h_pend[hh] = pltpu.async_copy(
                obuf.at[hh], out_h.at[pl.ds(orow, KH)], osems[hh]
            )
    for hh in range(2):
        out_h_pend[hh].wait()


@jax.jit
def _tt_gather(row, col, t3):
    mesh = plsc.VectorSubcoreMesh(core_axis_name="c", subcore_axis_name="s")
    f = pl.kernel(
        _body,
        mesh=mesh,
        out_type=jax.ShapeDtypeStruct((R1 * B, B * R2), jnp.float32),
        compiler_params=pltpu.CompilerParams(
            needs_layout_passes=False, use_tc_tiling_on_sc=False
        ),
        scratch_types=[
            pltpu.VMEM((RPW,), jnp.int32),
            pltpu.VMEM((B,), jnp.int32),
            pltpu.VMEM((R1 * RPW,), jnp.int32),
            pltpu.VMEM((B * R2,), jnp.int32),
            pltpu.VMEM((2, K, W), jnp.float32),
            pltpu.VMEM((2, KH, B * R2), jnp.float32),
            pltpu.SemaphoreType.DMA,
            pltpu.SemaphoreType.DMA,
            pltpu.SemaphoreType.DMA,
            pltpu.SemaphoreType.DMA,
        ],
    )
    return f(row, col, t3)


def kernel(indices, TT_core):
    row = indices[:, 0].astype(jnp.int32)
    col = indices[:, 1].astype(jnp.int32)
    t3 = TT_core.reshape(R1 * N1, W)
    out = _tt_gather(row, col, t3)
    return out.reshape(R1, B, B, R2)


# speedup vs baseline: 7.5913x; 6.6362x over previous
import jax
import jax.numpy as jnp
from jax import lax
from jax.experimental import pallas as pl
from jax.experimental.pallas import tpu as pltpu
from jax.experimental.pallas import tpu_sc as plsc

R1, N1, N2, R2 = 4, 1000, 1000, 4
B = 1024
NC, NS, L = 2, 16, 16
NW = NC * NS
RPW = B // NW
K = 8
KH = K // 2
G = (B * R2) // L
W = R2 * N2
NCHUNK = R1 * RPW // K


def _body(row_h, col_h, t3_h, out_h, rbuf, cbuf, ridx, cidx, ibuf, obuf,
          isem0, isem1, osem0, osem1):
    wid = lax.axis_index("s") * NC + lax.axis_index("c")
    i0 = wid * RPW
    isems = (isem0, isem1)
    osems = (osem0, osem1)

    pltpu.sync_copy(row_h.at[pl.ds(i0, RPW)], rbuf)

    for r1 in range(R1):
        for h in range(RPW // L):
            ridx[pl.ds(r1 * RPW + h * L, L)] = rbuf[pl.ds(h * L, L)] + r1 * N1

    def start_in(ci, slot):
        return pltpu.async_copy(
            t3_h.at[ridx.at[pl.ds(ci * K, K)]], ibuf.at[slot], isems[slot]
        )

    in_h = [None, None]
    in_h[0] = start_in(0, 0)

    pltpu.sync_copy(col_h, cbuf)

    @plsc.parallel_loop(0, G, unroll=2)
    def _cbody(g):
        jt = g >> 5
        r2 = (g >> 3) & 3
        j0 = (jt << 7) + ((g & 7) << 4)
        cidx[pl.ds(g * L, L)] = cbuf[pl.ds(j0, L)] + r2 * N2

    out_h_pend = [None, None]
    for ci in range(NCHUNK):
        slot = ci % 2
        if ci + 1 < NCHUNK:
            in_h[1 - slot] = start_in(ci + 1, 1 - slot)
        in_h[slot].wait()
        r1, cc = ci // (RPW // K), ci % (RPW // K)
        for hh in range(2):
            if out_h_pend[hh] is not None:
                out_h_pend[hh].wait()
            src = ibuf.at[slot]
            dst = obuf.at[hh]

            @plsc.parallel_loop(0, G, unroll=2)
            def _gbody(g):
                cvec = cidx[pl.ds(g * L, L)]
                for s in range(KH):
                    dst[s, pl.ds(g * L, L)] = plsc.load_gather(
                        src.at[hh * KH + s], [cvec]
                    )

            orow = r1 * B + i0 + cc * K + hh * KH
            out_h_pend[hh] = pltpu.async_copy(
                obuf.at[hh], out_h.at[pl.ds(orow, KH)], osems[hh]
            )
    for hh in range(2):
        out_h_pend[hh].wait()


@jax.jit
def _tt_gather(row, col, t3):
    mesh = plsc.VectorSubcoreMesh(core_axis_name="c", subcore_axis_name="s")
    f = pl.kernel(
        _body,
        mesh=mesh,
        out_type=jax.ShapeDtypeStruct((R1 * B, B * R2), jnp.float32),
        compiler_params=pltpu.CompilerParams(
            needs_layout_passes=False, use_tc_tiling_on_sc=False
        ),
        scratch_types=[
            pltpu.VMEM((RPW,), jnp.int32),
            pltpu.VMEM((B,), jnp.int32),
            pltpu.VMEM((R1 * RPW,), jnp.int32),
            pltpu.VMEM((B * R2,), jnp.int32),
            pltpu.VMEM((2, K, W), jnp.float32),
            pltpu.VMEM((2, KH, B * R2), jnp.float32),
            pltpu.SemaphoreType.DMA,
            pltpu.SemaphoreType.DMA,
            pltpu.SemaphoreType.DMA,
            pltpu.SemaphoreType.DMA,
        ],
    )
    return f(row, col, t3)


def kernel(indices, TT_core):
    row = indices[:, 0].astype(jnp.int32)
    col = indices[:, 1].astype(jnp.int32)
    t3 = TT_core.transpose(0, 1, 3, 2).reshape(R1 * N1, W)
    out = _tt_gather(row, col, t3)
    out = out.reshape(R1, B, B // 128, R2, 128)
    out = out.transpose(0, 1, 2, 4, 3).reshape(R1, B, B, R2)
    return out


# speedup vs baseline: 7.5955x; 1.0005x over previous
import jax
import jax.numpy as jnp
from jax import lax
from jax.experimental import pallas as pl
from jax.experimental.pallas import tpu as pltpu
from jax.experimental.pallas import tpu_sc as plsc

R1, N1, N2, R2 = 4, 1000, 1000, 4
B = 1024
NC, NS, L = 2, 16, 16
NW = NC * NS
RPW = B // NW
K = 8
KH = K // 2
G = (B * R2) // L
W = R2 * N2
NCHUNK = R1 * RPW // K


def _body(row_h, col_h, t3_h, out_h, rbuf, cbuf, ridx, cidx, ibuf, obuf,
          isem0, isem1, osem0, osem1):
    wid = lax.axis_index("s") * NC + lax.axis_index("c")
    i0 = wid * RPW
    isems = (isem0, isem1)
    osems = (osem0, osem1)

    pltpu.sync_copy(row_h.at[pl.ds(i0, RPW)], rbuf)

    for r1 in range(R1):
        for h in range(RPW // L):
            ridx[pl.ds(r1 * RPW + h * L, L)] = rbuf[pl.ds(h * L, L)] + r1 * N1

    def start_in(ci, slot):
        return pltpu.async_copy(
            t3_h.at[ridx.at[pl.ds(ci * K, K)]], ibuf.at[slot], isems[slot]
        )

    in_h = [None, None]
    in_h[0] = start_in(0, 0)

    pltpu.sync_copy(col_h, cbuf)

    @plsc.parallel_loop(0, G, unroll=2)
    def _cbody(g):
        jt = g >> 5
        r2 = (g >> 3) & 3
        j0 = (jt << 7) + ((g & 7) << 4)
        cidx[pl.ds(g * L, L)] = cbuf[pl.ds(j0, L)] + r2 * N2

    out_h_pend = [None, None]
    for ci in range(NCHUNK):
        slot = ci % 2
        if ci + 1 < NCHUNK:
            in_h[1 - slot] = start_in(ci + 1, 1 - slot)
        in_h[slot].wait()
        r1, cc = ci // (RPW // K), ci % (RPW // K)
        for hh in range(2):
            if out_h_pend[hh] is not None:
                out_h_pend[hh].wait()
            src = ibuf.at[slot]
            dst = obuf.at[hh]

            @plsc.parallel_loop(0, G, unroll=4)
            def _gbody(g):
                cvec = cidx[pl.ds(g * L, L)]
                for s in range(KH):
                    dst[s, pl.ds(g * L, L)] = plsc.load_gather(
                        src.at[hh * KH + s], [cvec]
                    )

            orow = r1 * B + i0 + cc * K + hh * KH
            out_h_pend[hh] = pltpu.async_copy(
                obuf.at[hh], out_h.at[pl.ds(orow, KH)], osems[hh]
            )
    for hh in range(2):
        out_h_pend[hh].wait()


@jax.jit
def _tt_gather(row, col, t3):
    mesh = plsc.VectorSubcoreMesh(core_axis_name="c", subcore_axis_name="s")
    f = pl.kernel(
        _body,
        mesh=mesh,
        out_type=jax.ShapeDtypeStruct((R1 * B, B * R2), jnp.float32),
        compiler_params=pltpu.CompilerParams(
            needs_layout_passes=False, use_tc_tiling_on_sc=False
        ),
        scratch_types=[
            pltpu.VMEM((RPW,), jnp.int32),
            pltpu.VMEM((B,), jnp.int32),
            pltpu.VMEM((R1 * RPW,), jnp.int32),
            pltpu.VMEM((B * R2,), jnp.int32),
            pltpu.VMEM((2, K, W), jnp.float32),
            pltpu.VMEM((2, KH, B * R2), jnp.float32),
            pltpu.SemaphoreType.DMA,
            pltpu.SemaphoreType.DMA,
            pltpu.SemaphoreType.DMA,
            pltpu.SemaphoreType.DMA,
        ],
    )
    return f(row, col, t3)


def kernel(indices, TT_core):
    row = indices[:, 0].astype(jnp.int32)
    col = indices[:, 1].astype(jnp.int32)
    t3 = TT_core.transpose(0, 1, 3, 2).reshape(R1 * N1, W)
    out = _tt_gather(row, col, t3)
    out = out.reshape(R1, B, B // 128, R2, 128)
    out = out.transpose(0, 1, 2, 4, 3).reshape(R1, B, B, R2)
    return out


# speedup vs baseline: 7.6020x; 1.0009x over previous
import jax
import jax.numpy as jnp
from jax import lax
from jax.experimental import pallas as pl
from jax.experimental.pallas import tpu as pltpu
from jax.experimental.pallas import tpu_sc as plsc

R1, N1, N2, R2 = 4, 1000, 1000, 4
B = 1024
NC, NS, L = 2, 16, 16
NW = NC * NS
RPW = B // NW
K = 8
G = (B * R2) // L
W = R2 * N2
NCHUNK = R1 * RPW // K


def _body(row_h, col_h, t3_h, out_h, rbuf, cbuf, ridx, ibuf, obuf,
          isem0, isem1, osem0, osem1):
    wid = lax.axis_index("s") * NC + lax.axis_index("c")
    i0 = wid * RPW
    isems = (isem0, isem1)
    osems = (osem0, osem1)

    pltpu.sync_copy(row_h.at[pl.ds(i0, RPW)], rbuf)

    for r1 in range(R1):
        for h in range(RPW // L):
            ridx[pl.ds(r1 * RPW + h * L, L)] = rbuf[pl.ds(h * L, L)] + r1 * N1

    def start_in(ci, slot):
        return pltpu.async_copy(
            t3_h.at[ridx.at[pl.ds(ci * K, K)]], ibuf.at[slot], isems[slot]
        )

    in_h = [None, None]
    in_h[0] = start_in(0, 0)
    pltpu.sync_copy(col_h, cbuf)

    out_pend = [None, None]
    for ci in range(NCHUNK):
        slot = ci % 2
        if ci + 1 < NCHUNK:
            in_h[1 - slot] = start_in(ci + 1, 1 - slot)
        in_h[slot].wait()
        if out_pend[slot] is not None:
            out_pend[slot].wait()
        src = ibuf.at[slot]
        dst = obuf.at[slot]

        @plsc.parallel_loop(0, G, unroll=2)
        def _gbody(g):
            jt = g >> 5
            r2 = (g >> 3) & 3
            j0 = (jt << 7) + ((g & 7) << 4)
            cvec = cbuf[pl.ds(j0, L)] + r2 * N2
            for s in range(K):
                dst[s, pl.ds(g * L, L)] = plsc.load_gather(src.at[s], [cvec])

        r1, cc = ci // (RPW // K), ci % (RPW // K)
        orow = r1 * B + i0 + cc * K
        out_pend[slot] = pltpu.async_copy(
            obuf.at[slot], out_h.at[pl.ds(orow, K)], osems[slot]
        )
    for slot in range(2):
        out_pend[slot].wait()


@jax.jit
def _tt_gather(row, col, t3):
    mesh = plsc.VectorSubcoreMesh(core_axis_name="c", subcore_axis_name="s")
    f = pl.kernel(
        _body,
        mesh=mesh,
        out_type=jax.ShapeDtypeStruct((R1 * B, B * R2), jnp.float32),
        compiler_params=pltpu.CompilerParams(
            needs_layout_passes=False, use_tc_tiling_on_sc=False
        ),
        scratch_types=[
            pltpu.VMEM((RPW,), jnp.int32),
            pltpu.VMEM((B,), jnp.int32),
            pltpu.VMEM((R1 * RPW,), jnp.int32),
            pltpu.VMEM((2, K, W), jnp.float32),
            pltpu.VMEM((2, K, B * R2), jnp.float32),
            pltpu.SemaphoreType.DMA,
            pltpu.SemaphoreType.DMA,
            pltpu.SemaphoreType.DMA,
            pltpu.SemaphoreType.DMA,
        ],
    )
    return f(row, col, t3)


def kernel(indices, TT_core):
    row = indices[:, 0].astype(jnp.int32)
    col = indices[:, 1].astype(jnp.int32)
    t3 = TT_core.transpose(0, 1, 3, 2).reshape(R1 * N1, W)
    out = _tt_gather(row, col, t3)
    out = out.reshape(R1, B, B // 128, R2, 128)
    out = out.transpose(0, 1, 2, 4, 3).reshape(R1, B, B, R2)
    return out
